# Initial kernel scaffold; baseline (speedup 1.0000x reference)
import jax
import jax.numpy as jnp
from jax.experimental import pallas as pl


def kernel(x, Wg, bg, Wu, Wv, Wd):
    raise NotImplementedError("write your pallas kernel here")


# speedup vs baseline: 1.8980x; 1.8980x over previous
import functools

import jax
import jax.numpy as jnp
from jax.experimental import pallas as pl
from jax.experimental.pallas import tpu as pltpu

B, T, D, E = 2, 2048, 1024, 8
H = 2752
ALPHA = 0.05
N = B * T

BT = 256
NH = 4
HT = H // NH
G = N // BT + E
P = G * BT


def _ffn_kernel(be_ref, xs_ref, wu_ref, wv_ref, wd_ref, p_ref, out_ref):
    xb = xs_ref[...].astype(jnp.bfloat16)
    dn = (((1,), (1,)), ((), ()))
    u = jax.lax.dot_general(xb, wu_ref[0], dn, preferred_element_type=jnp.float32)
    v = jax.lax.dot_general(xb, wv_ref[0], dn, preferred_element_type=jnp.float32)
    act = (u * jax.nn.sigmoid(u) * v).astype(jnp.bfloat16)
    y = jax.lax.dot_general(act, wd_ref[0], dn, preferred_element_type=jnp.float32)
    out_ref[...] = y * p_ref[...]


def _grouped_ffn(xs, wu, wv, wd, p_sorted, block_expert):
    grid_spec = pltpu.PrefetchScalarGridSpec(
        num_scalar_prefetch=1,
        grid=(G,),
        in_specs=[
            pl.BlockSpec((BT, D), lambda g, be: (g, 0)),
            pl.BlockSpec((1, H, D), lambda g, be: (be[g], 0, 0)),
            pl.BlockSpec((1, H, D), lambda g, be: (be[g], 0, 0)),
            pl.BlockSpec((1, D, H), lambda g, be: (be[g], 0, 0)),
            pl.BlockSpec((BT, 1), lambda g, be: (g, 0)),
        ],
        out_specs=pl.BlockSpec((BT, D), lambda g, be: (g, 0)),
    )
    return pl.pallas_call(
        _ffn_kernel,
        grid_spec=grid_spec,
        out_shape=jax.ShapeDtypeStruct((P, D), jnp.float32),
        compiler_params=pltpu.CompilerParams(
            dimension_semantics=("arbitrary",),
        ),
    )(block_expert, xs, wu, wv, wd, p_sorted)


@jax.jit
def kernel(x, Wg, bg, Wu, Wv, Wd):
    xf = x.reshape(N, D)

    logits = xf @ Wg.T + bg
    probs = jax.nn.softmax(logits, axis=-1)
    top1_idx = jnp.argmax(logits, axis=-1)
    top1_p = jnp.take_along_axis(probs, top1_idx[:, None], axis=-1)[:, 0]
    one_hot = jax.nn.one_hot(top1_idx, E, dtype=jnp.float32)
    me = jax.lax.stop_gradient(one_hot.mean(axis=0))
    ce = jax.lax.stop_gradient(probs.mean(axis=0))
    aux = ALPHA * E * jnp.sum(me * ce)

    counts = jnp.sum(one_hot, axis=0).astype(jnp.int32)
    rank = (jnp.cumsum(one_hot, axis=0) - one_hot)
    rank = jnp.take_along_axis(rank, top1_idx[:, None], axis=-1)[:, 0]
    rank = rank.astype(jnp.int32)
    padded = ((counts + BT - 1) // BT) * BT
    pad_start = jnp.concatenate([jnp.zeros((1,), jnp.int32),
                                 jnp.cumsum(padded)[:-1]])
    pad_end = jnp.cumsum(padded)
    pos = pad_start[top1_idx] + rank
    src_idx = jnp.zeros((P,), jnp.int32).at[pos].set(
        jnp.arange(N, dtype=jnp.int32))
    blk_starts = jnp.arange(G, dtype=jnp.int32) * BT
    block_expert = jnp.minimum(
        jnp.sum(blk_starts[:, None] >= pad_end[None, :], axis=1), E - 1
    ).astype(jnp.int32)

    xs = xf[src_idx]
    p_sorted = top1_p[src_idx][:, None]

    wu16 = Wu.astype(jnp.bfloat16)
    wv16 = Wv.astype(jnp.bfloat16)
    wd16 = Wd.astype(jnp.bfloat16)
    out_sorted = _grouped_ffn(xs, wu16, wv16, wd16, p_sorted, block_expert)

    y = out_sorted[pos]
    return y.reshape(B, T, D), aux
